# baseline (speedup 1.0000x reference)
import jax
import jax.numpy as jnp
from jax.experimental import pallas as pl

_B, _N, _D_FEAT, _HID, _LAYERS = 8, 256, 256, 256, 3
_N_TYPES, _N_CLASSES = 10, 8
_NT_PAD = 16
_C_PAD = 128


def _gnn_body(nf_ref, adj_ref, nt_ref, emb_ref, projwt_ref, projb_ref,
              linwt_ref, linb_ref, attw_ref, attb_ref,
              poolw1t_ref, poolb1_ref, poolw2t_ref, poolb2_ref,
              clsw1t_ref, clsb1_ref, clsw2t_ref, clsb2_ref,
              scores_ref, gemb_ref):
    f32 = jnp.float32
    x = nf_ref[0]
    types = nt_ref[0]
    iota = jax.lax.broadcasted_iota(jnp.int32, (_N, _NT_PAD), 1)
    onehot = (iota == types).astype(f32)
    type_emb = jnp.dot(onehot, emb_ref[...], preferred_element_type=f32)
    feat = jnp.dot(x, projwt_ref[...], preferred_element_type=f32)
    h = type_emb + feat + projb_ref[...]
    mask = (adj_ref[0] > 0.0).astype(f32)
    for l in range(_LAYERS):
        t = jnp.dot(h, linwt_ref[l], preferred_element_type=f32) + linb_ref[l:l + 1]
        s12 = jnp.dot(t, attw_ref[l], preferred_element_type=f32)
        s1 = s12[:, 0:1]
        s2 = s12[:, 1:2]
        logits = s1 + s2.T + attb_ref[l:l + 1]
        w = jax.nn.sigmoid(logits) * mask
        h = jax.nn.relu(t + jnp.dot(w, t, preferred_element_type=f32))
    hp = jnp.tanh(jnp.dot(h, poolw1t_ref[...], preferred_element_type=f32)
                  + poolb1_ref[...])
    a = jnp.dot(hp, poolw2t_ref[...], preferred_element_type=f32) + poolb2_ref[...]
    a = jax.nn.softmax(a, axis=0)
    g = jnp.dot(a.T, h, preferred_element_type=f32)
    z = jax.nn.relu(jnp.dot(g, clsw1t_ref[...], preferred_element_type=f32)
                    + clsb1_ref[...])
    s = jnp.dot(z, clsw2t_ref[...], preferred_element_type=f32) + clsb2_ref[...]
    scores_ref[0] = s
    gemb_ref[0] = g


def kernel(node_features, adjacency, node_types, emb_table, proj_w, proj_b,
           lin_w, lin_b, att_w, att_b, pool_w1, pool_b1, pool_w2, pool_b2,
           cls_w1, cls_b1, cls_w2, cls_b2):
    f32 = jnp.float32
    nt3 = node_types.astype(jnp.int32).reshape(_B, _N, 1)
    emb_pad = jnp.zeros((_NT_PAD, _HID), f32).at[:_N_TYPES].set(emb_table)
    proj_wt = proj_w.T
    lin_wt = jnp.swapaxes(lin_w, 1, 2)
    att_wt = jnp.swapaxes(att_w[:, 0, :].reshape(_LAYERS, 2, _HID), 1, 2)
    pool_w1t = pool_w1.T
    pool_w2t = pool_w2.T
    cls_w1t = cls_w1.T
    cls_w2t = jnp.zeros((_HID // 2, _C_PAD), f32).at[:, :_N_CLASSES].set(cls_w2.T)
    cls_b2p = jnp.zeros((1, _C_PAD), f32).at[0, :_N_CLASSES].set(cls_b2)

    def full(shape):
        n = len(shape)
        return pl.BlockSpec(shape, lambda b, _n=n: (0,) * _n)

    grid_spec = pl.GridSpec(
        grid=(_B,),
        in_specs=[
            pl.BlockSpec((1, _N, _D_FEAT), lambda b: (b, 0, 0)),
            pl.BlockSpec((1, _N, _N), lambda b: (0, 0, 0)),
            pl.BlockSpec((1, _N, 1), lambda b: (b, 0, 0)),
            full((_NT_PAD, _HID)),
            full((_D_FEAT, _HID)),
            full((1, _HID)),
            full((_LAYERS, _HID, _HID)),
            full((_LAYERS, _HID)),
            full((_LAYERS, _HID, 2)),
            full((_LAYERS, 1)),
            full((_HID, _HID // 2)),
            full((1, _HID // 2)),
            full((_HID // 2, 1)),
            full((1, 1)),
            full((_HID, _HID // 2)),
            full((1, _HID // 2)),
            full((_HID // 2, _C_PAD)),
            full((1, _C_PAD)),
        ],
        out_specs=[
            pl.BlockSpec((1, 1, _C_PAD), lambda b: (b, 0, 0)),
            pl.BlockSpec((1, 1, _HID), lambda b: (b, 0, 0)),
        ],
    )
    scores_pad, gemb = pl.pallas_call(
        _gnn_body,
        grid_spec=grid_spec,
        out_shape=[
            jax.ShapeDtypeStruct((_B, 1, _C_PAD), f32),
            jax.ShapeDtypeStruct((_B, 1, _HID), f32),
        ],
    )(node_features, adjacency, nt3, emb_pad, proj_wt,
      proj_b.reshape(1, _HID), lin_wt, lin_b, att_wt, att_b,
      pool_w1t, pool_b1.reshape(1, _HID // 2), pool_w2t,
      pool_b2.reshape(1, 1), cls_w1t, cls_b1.reshape(1, _HID // 2),
      cls_w2t, cls_b2p)
    return scores_pad[:, 0, :_N_CLASSES], gemb[:, 0, :]


# speedup vs baseline: 1.4325x; 1.4325x over previous
import jax
import jax.numpy as jnp
from jax.experimental import pallas as pl

_B, _N, _D_FEAT, _HID, _LAYERS = 8, 256, 256, 256, 3
_N_TYPES, _N_CLASSES = 10, 8
_NT_PAD = 16
_C_PAD = 128


def _gnn_body(nf_ref, adj_ref, nt_ref, emb_ref, projwt_ref, projb_ref,
              linwt_ref, linb_ref, attw_ref, attb_ref,
              poolw1t_ref, poolb1_ref, poolw2t_ref, poolb2_ref,
              clsw1t_ref, clsb1_ref, clsw2t_ref, clsb2_ref,
              scores_ref, gemb_ref):
    f32 = jnp.float32
    bn = _B * _N
    x = nf_ref[...]
    types = nt_ref[...]
    iota = jax.lax.broadcasted_iota(jnp.int32, (bn, _NT_PAD), 1)
    onehot = (iota == types).astype(f32)
    type_emb = jnp.dot(onehot, emb_ref[...], preferred_element_type=f32)
    feat = jnp.dot(x, projwt_ref[...], preferred_element_type=f32)
    h = type_emb + feat + projb_ref[...]
    mask = (adj_ref[0] > 0.0).astype(f32)
    for l in range(_LAYERS):
        t = jnp.dot(h, linwt_ref[l], preferred_element_type=f32) + linb_ref[l:l + 1]
        s12 = jnp.dot(t, attw_ref[l], preferred_element_type=f32)
        blocks = []
        for b in range(_B):
            tb = t[b * _N:(b + 1) * _N]
            s1 = s12[b * _N:(b + 1) * _N, 0:1]
            s2 = s12[b * _N:(b + 1) * _N, 1:2]
            logits = s1 + s2.T + attb_ref[l:l + 1]
            w = jax.nn.sigmoid(logits) * mask
            blocks.append(jax.nn.relu(tb + jnp.dot(w, tb, preferred_element_type=f32)))
        h = jnp.concatenate(blocks, axis=0)
    hp = jnp.tanh(jnp.dot(h, poolw1t_ref[...], preferred_element_type=f32)
                  + poolb1_ref[...])
    a = jnp.dot(hp, poolw2t_ref[...], preferred_element_type=f32) + poolb2_ref[...]
    gs = []
    for b in range(_B):
        ab = jax.nn.softmax(a[b * _N:(b + 1) * _N], axis=0)
        gs.append(jnp.dot(ab.T, h[b * _N:(b + 1) * _N],
                          preferred_element_type=f32))
    g = jnp.concatenate(gs, axis=0)
    z = jax.nn.relu(jnp.dot(g, clsw1t_ref[...], preferred_element_type=f32)
                    + clsb1_ref[...])
    s = jnp.dot(z, clsw2t_ref[...], preferred_element_type=f32) + clsb2_ref[...]
    scores_ref[...] = s
    gemb_ref[...] = g


def kernel(node_features, adjacency, node_types, emb_table, proj_w, proj_b,
           lin_w, lin_b, att_w, att_b, pool_w1, pool_b1, pool_w2, pool_b2,
           cls_w1, cls_b1, cls_w2, cls_b2):
    f32 = jnp.float32
    nf2 = node_features.reshape(_B * _N, _D_FEAT)
    nt2 = node_types.astype(jnp.int32).reshape(_B * _N, 1)
    adj0 = adjacency[:1, :, :]
    emb_pad = jnp.zeros((_NT_PAD, _HID), f32).at[:_N_TYPES].set(emb_table)
    proj_wt = proj_w.T
    lin_wt = jnp.swapaxes(lin_w, 1, 2)
    att_wt = jnp.swapaxes(att_w[:, 0, :].reshape(_LAYERS, 2, _HID), 1, 2)
    pool_w1t = pool_w1.T
    pool_w2t = pool_w2.T
    cls_w1t = cls_w1.T
    cls_w2t = jnp.zeros((_HID // 2, _C_PAD), f32).at[:, :_N_CLASSES].set(cls_w2.T)
    cls_b2p = jnp.zeros((1, _C_PAD), f32).at[0, :_N_CLASSES].set(cls_b2)

    scores_pad, gemb = pl.pallas_call(
        _gnn_body,
        out_shape=[
            jax.ShapeDtypeStruct((_B, _C_PAD), f32),
            jax.ShapeDtypeStruct((_B, _HID), f32),
        ],
    )(nf2, adj0, nt2, emb_pad, proj_wt,
      proj_b.reshape(1, _HID), lin_wt, lin_b, att_wt, att_b,
      pool_w1t, pool_b1.reshape(1, _HID // 2), pool_w2t,
      pool_b2.reshape(1, 1), cls_w1t, cls_b1.reshape(1, _HID // 2),
      cls_w2t, cls_b2p)
    return scores_pad[:, :_N_CLASSES], gemb


# speedup vs baseline: 2.5919x; 1.8094x over previous
import jax
import jax.numpy as jnp
from jax.experimental import pallas as pl

_B, _N, _D_FEAT, _HID, _LAYERS = 8, 256, 256, 256, 3
_N_TYPES, _N_CLASSES = 10, 8
_NT_PAD = 16


def _dgt(x, w):
    return jax.lax.dot_general(x, w, (((1,), (1,)), ((), ())),
                               preferred_element_type=jnp.float32)


def _gnn_body(nf_ref, adj_ref, nt_ref, emb_ref, projw_ref, projb_ref,
              linw_ref, linb_ref, attw_ref, attb_ref,
              poolw1_ref, poolb1_ref, poolw2_ref,
              clsw1_ref, clsb1_ref, clsw2_ref, clsb2_ref,
              scores_ref, gemb_ref):
    f32 = jnp.float32
    bn = _B * _N
    x = nf_ref[...]
    types = nt_ref[...]
    iota = jax.lax.broadcasted_iota(jnp.int32, (bn, _NT_PAD), 1)
    onehot = (iota == types).astype(f32)
    type_emb = jnp.dot(onehot, emb_ref[...], preferred_element_type=f32)
    h = type_emb + _dgt(x, projw_ref[...]) + projb_ref[...]
    mask = (adj_ref[0] > 0.0).astype(f32)
    for l in range(_LAYERS):
        t = _dgt(h, linw_ref[l]) + linb_ref[l:l + 1]
        aw = attw_ref[l:l + 1]
        w1 = aw[:, :_HID]
        w2 = aw[:, _HID:]
        s1 = _dgt(t, w1)
        blocks = []
        for b in range(_B):
            tb = t[b * _N:(b + 1) * _N]
            s2 = _dgt(w2, tb)
            logits = s1[b * _N:(b + 1) * _N] + s2 + attb_ref[l:l + 1]
            w = jax.nn.sigmoid(logits) * mask
            blocks.append(jax.nn.relu(tb + jnp.dot(w, tb, preferred_element_type=f32)))
        h = jnp.concatenate(blocks, axis=0)
    hp = jnp.tanh(_dgt(h, poolw1_ref[...]) + poolb1_ref[...])
    a = _dgt(hp, poolw2_ref[...])
    gs = []
    for b in range(_B):
        ab = jax.nn.softmax(a[b * _N:(b + 1) * _N], axis=0)
        hb = h[b * _N:(b + 1) * _N]
        gs.append(jax.lax.dot_general(ab, hb, (((0,), (0,)), ((), ())),
                                      preferred_element_type=f32))
    g = jnp.concatenate(gs, axis=0)
    z = jax.nn.relu(_dgt(g, clsw1_ref[...]) + clsb1_ref[...])
    scores_ref[...] = _dgt(z, clsw2_ref[...]) + clsb2_ref[...]
    gemb_ref[...] = g


def kernel(node_features, adjacency, node_types, emb_table, proj_w, proj_b,
           lin_w, lin_b, att_w, att_b, pool_w1, pool_b1, pool_w2, pool_b2,
           cls_w1, cls_b1, cls_w2, cls_b2):
    f32 = jnp.float32
    nf2 = node_features.reshape(_B * _N, _D_FEAT)
    nt2 = node_types.astype(jnp.int32).reshape(_B * _N, 1)
    emb_pad = jnp.zeros((_NT_PAD, _HID), f32).at[:_N_TYPES].set(emb_table)

    def full(shape):
        n = len(shape)
        return pl.BlockSpec(shape, lambda i, _n=n: (0,) * _n)

    scores, gemb = pl.pallas_call(
        _gnn_body,
        grid=(1,),
        in_specs=[
            full((_B * _N, _D_FEAT)),
            pl.BlockSpec((1, _N, _N), lambda i: (0, 0, 0)),
            full((_B * _N, 1)),
            full((_NT_PAD, _HID)),
            full((_HID, _D_FEAT)),
            full((1, _HID)),
            full((_LAYERS, _HID, _HID)),
            full((_LAYERS, _HID)),
            full((_LAYERS, 2 * _HID)),
            full((_LAYERS, 1)),
            full((_HID // 2, _HID)),
            full((1, _HID // 2)),
            full((1, _HID // 2)),
            full((_HID // 2, _HID)),
            full((1, _HID // 2)),
            full((_N_CLASSES, _HID // 2)),
            full((1, _N_CLASSES)),
        ],
        out_specs=[
            full((_B, _N_CLASSES)),
            full((_B, _HID)),
        ],
        out_shape=[
            jax.ShapeDtypeStruct((_B, _N_CLASSES), f32),
            jax.ShapeDtypeStruct((_B, _HID), f32),
        ],
    )(nf2, adjacency, nt2, emb_pad, proj_w,
      proj_b.reshape(1, _HID), lin_w, lin_b, att_w[:, 0, :], att_b,
      pool_w1, pool_b1.reshape(1, _HID // 2), pool_w2,
      cls_w1, cls_b1.reshape(1, _HID // 2),
      cls_w2, cls_b2.reshape(1, _N_CLASSES))
    return scores, gemb


# speedup vs baseline: 3.6633x; 1.4134x over previous
import jax
import jax.numpy as jnp
from jax.experimental import pallas as pl

_B, _N, _D_FEAT, _HID, _LAYERS = 8, 256, 256, 256, 3
_N_TYPES, _N_CLASSES = 10, 8


def _dgt(x, w):
    return jax.lax.dot_general(x, w, (((1,), (1,)), ((), ())),
                               preferred_element_type=jnp.float32)


def _gnn_body(nf_ref, adj_ref, nt_ref, emb_ref, projw_ref, projb_ref,
              linw_ref, linb_ref, attw_ref, attb_ref,
              poolw1_ref, poolb1_ref, poolw2_ref,
              clsw1_ref, clsb1_ref, clsw2_ref, clsb2_ref,
              scores_ref, gemb_ref):
    f32 = jnp.float32
    x = nf_ref[...].reshape(_B * _N, _D_FEAT)
    feat = _dgt(x, projw_ref[...]) + projb_ref[...].reshape(1, _HID)
    kiota = jax.lax.broadcasted_iota(jnp.int32, (_N_TYPES, _N), 0)
    hb_list = []
    for b in range(_B):
        ohT = (kiota == nt_ref[b:b + 1, :]).astype(f32)
        te = jax.lax.dot_general(ohT, emb_ref[...], (((0,), (0,)), ((), ())),
                                 preferred_element_type=f32)
        hb_list.append(feat[b * _N:(b + 1) * _N] + te)
    h = jnp.concatenate(hb_list, axis=0)
    mask = (adj_ref[0] > 0.0).astype(f32)
    for l in range(_LAYERS):
        t = _dgt(h, linw_ref[l]) + linb_ref[l:l + 1]
        aw = attw_ref[l]
        w1 = aw[:, :_HID]
        w2 = aw[:, _HID:]
        s1 = _dgt(t, w1)
        blocks = []
        for b in range(_B):
            tb = t[b * _N:(b + 1) * _N]
            s2 = _dgt(w2, tb)
            logits = s1[b * _N:(b + 1) * _N] + s2 + attb_ref[l:l + 1]
            w = jax.nn.sigmoid(logits) * mask
            blocks.append(jax.nn.relu(tb + jnp.dot(w, tb, preferred_element_type=f32)))
        h = jnp.concatenate(blocks, axis=0)
    hp = jnp.tanh(_dgt(h, poolw1_ref[...]) + poolb1_ref[...].reshape(1, _HID // 2))
    a = _dgt(hp, poolw2_ref[...])
    gs = []
    for b in range(_B):
        ab = jax.nn.softmax(a[b * _N:(b + 1) * _N], axis=0)
        hb = h[b * _N:(b + 1) * _N]
        gs.append(jax.lax.dot_general(ab, hb, (((0,), (0,)), ((), ())),
                                      preferred_element_type=f32))
    g = jnp.concatenate(gs, axis=0)
    z = jax.nn.relu(_dgt(g, clsw1_ref[...]) + clsb1_ref[...].reshape(1, _HID // 2))
    scores_ref[...] = _dgt(z, clsw2_ref[...]) + clsb2_ref[...].reshape(1, _N_CLASSES)
    gemb_ref[...] = g


def kernel(node_features, adjacency, node_types, emb_table, proj_w, proj_b,
           lin_w, lin_b, att_w, att_b, pool_w1, pool_b1, pool_w2, pool_b2,
           cls_w1, cls_b1, cls_w2, cls_b2):
    f32 = jnp.float32
    del pool_b2

    def full(shape):
        n = len(shape)
        return pl.BlockSpec(shape, lambda i, _n=n: (0,) * _n)

    scores, gemb = pl.pallas_call(
        _gnn_body,
        grid=(1,),
        in_specs=[
            full((_B, _N, _D_FEAT)),
            pl.BlockSpec((1, _N, _N), lambda i: (0, 0, 0)),
            full((_B, _N)),
            full((_N_TYPES, _HID)),
            full((_HID, _D_FEAT)),
            full((_HID,)),
            full((_LAYERS, _HID, _HID)),
            full((_LAYERS, _HID)),
            full((_LAYERS, 1, 2 * _HID)),
            full((_LAYERS, 1)),
            full((_HID // 2, _HID)),
            full((_HID // 2,)),
            full((1, _HID // 2)),
            full((_HID // 2, _HID)),
            full((_HID // 2,)),
            full((_N_CLASSES, _HID // 2)),
            full((_N_CLASSES,)),
        ],
        out_specs=[
            full((_B, _N_CLASSES)),
            full((_B, _HID)),
        ],
        out_shape=[
            jax.ShapeDtypeStruct((_B, _N_CLASSES), f32),
            jax.ShapeDtypeStruct((_B, _HID), f32),
        ],
    )(node_features, adjacency, node_types.astype(jnp.int32), emb_table,
      proj_w, proj_b, lin_w, lin_b, att_w, att_b,
      pool_w1, pool_b1, pool_w2, cls_w1, cls_b1, cls_w2, cls_b2)
    return scores, gemb
